# baseline (speedup 1.0000x reference)
import functools

import jax
import jax.numpy as jnp
from jax import lax
from jax.experimental import pallas as pl
from jax.experimental.pallas import tpu as pltpu
from jax.experimental.pallas import tpu_sc as plsc

VOCAB = 8192
N_IDX = 4 * 2048
NC = 2
NS = 16
NW = NC * NS
PER_W = N_IDX // NW
W = 1
NBUF = 8
A = NBUF // 2
NWIN = PER_W // W
NGRP = NWIN // NBUF

_mesh = plsc.VectorSubcoreMesh(core_axis_name="c", subcore_axis_name="s")


@functools.partial(
    pl.kernel,
    out_type=jax.ShapeDtypeStruct((N_IDX, VOCAB), jnp.float32),
    mesh=_mesh,
    scratch_types=[
        pltpu.VMEM((NWIN, W), jnp.int32),
        pltpu.VMEM((NBUF, W, VOCAB), jnp.float32),
        pltpu.SemaphoreType.DMA((NBUF,)),
        pltpu.SemaphoreType.DMA((NBUF,)),
    ],
)
def _lookup(idx_hbm, table_hbm, out_hbm, idx_v, rows_v, gsem, osem):
    wid = lax.axis_index("s") * NC + lax.axis_index("c")
    row0 = wid * PER_W

    pltpu.sync_copy(idx_hbm.at[wid], idx_v)

    def g_start(w, b):
        pltpu.async_copy(table_hbm.at[idx_v.at[w]], rows_v.at[b], gsem.at[b])

    def g_wait(w, b):
        pltpu.make_async_copy(
            table_hbm.at[idx_v.at[w]], rows_v.at[b], gsem.at[b]
        ).wait()

    def o_start(w, b):
        pltpu.async_copy(
            rows_v.at[b], out_hbm.at[pl.ds(row0 + w * W, W)], osem.at[b]
        )

    def o_wait(w, b):
        pltpu.make_async_copy(
            rows_v.at[b], out_hbm.at[pl.ds(row0 + w * W, W)], osem.at[b]
        ).wait()

    for b in range(A):
        g_start(b, b)
    for j in range(NBUF):
        b2 = (j + A) % NBUF
        if j >= A:
            o_wait(j - A, b2)
        g_start(j + A, b2)
        g_wait(j, j)
        o_start(j, j)

    def body(i, carry):
        for j in range(NBUF):
            w = i * NBUF + j
            b2 = (j + A) % NBUF
            o_wait(w - A, b2)
            g_start(w + A, b2)
            g_wait(w, j)
            o_start(w, j)
        return carry

    lax.fori_loop(1, NGRP - 1, body, 0)

    for j in range(NBUF):
        w = (NGRP - 1) * NBUF + j
        b2 = (j + A) % NBUF
        o_wait(w - A, b2)
        if w + A < NWIN:
            g_start(w + A, b2)
        g_wait(w, j)
        o_start(w, j)

    for j in range(NBUF - A, NBUF):
        w = NWIN - NBUF + j
        o_wait(w, j)


NB = 8
AD = NB // 2
NSLOT = PER_W + NB
NV = N_IDX // 16
SENT = PER_W << 13


@functools.partial(
    pl.kernel,
    out_type=jax.ShapeDtypeStruct((N_IDX, VOCAB), jnp.float32),
    mesh=_mesh,
    scratch_types=[
        pltpu.VMEM((N_IDX,), jnp.int32),
        pltpu.VMEM((N_IDX + 16,), jnp.int32),
        pltpu.VMEM((PER_W + 16,), jnp.int32),
        pltpu.VMEM((NSLOT + 16,), jnp.int32),
        pltpu.VMEM((NSLOT, 1), jnp.int32),
        pltpu.VMEM((N_IDX + 32,), jnp.int32),
        pltpu.VMEM((NB, 1, VOCAB), jnp.float32),
        pltpu.SemaphoreType.DMA((NB,)),
        pltpu.SemaphoreType.DMA((NB,)),
    ],
    compiler_params=pltpu.CompilerParams(needs_layout_passes=False),
)
def _lookup_dedup(idx_hbm, table_hbm, out_hbm,
                  idx1, hits, binc, ust, uniq, opos, rows, gsem, osem):
    i32 = jnp.int32
    wid = lax.axis_index("s") * NC + lax.axis_index("c")
    lo = wid * PER_W
    lanes = lax.iota(i32, 16)
    lane0 = lanes == 0
    zeros16 = jnp.zeros((16,), i32)

    def splat(x):
        return jnp.broadcast_to(x, (16,)).astype(i32)

    def ext(xv):
        return jnp.sum(jnp.where(lane0, xv, 0))

    def lane(xv, k):
        return jnp.sum(jnp.where(lanes == k, xv, 0))

    def rd(ref, i):
        return ext(plsc.load_gather(ref, (splat(i),)))

    pltpu.sync_copy(idx_hbm, idx1)

    def p1(v, off_v):
        x = plsc.load_gather(idx1, (v * 16 + lanes,))
        msk = jnp.logical_and(x >= lo, x < lo + PER_W)
        m32 = msk.astype(i32)
        cs = plsc.cumsum(m32)
        dst = off_v + cs - m32
        packed = jnp.bitwise_or(jnp.left_shift(x - lo, 13), v * 16 + lanes)
        plsc.store_scatter(hits, (dst,), packed, mask=msk)
        return off_v + plsc.all_reduce_population_count(msk)

    m_v = lax.fori_loop(0, NV, p1, zeros16)

    plsc.store_scatter(hits, (m_v + lanes,), jnp.full((16,), SENT, i32))

    for t in range((PER_W + 16) // 16):
        binc[pl.ds(t * 16, 16)] = zeros16

    m_s = ext(m_v)
    nch = (m_s + 15) // 16

    def pa(g, c):
        x = plsc.load_gather(hits, (g * 16 + lanes,))
        rall = lax.shift_right_logical(x, 13)
        for k in range(16):
            r_v = splat(lane(rall, k))
            c_v = plsc.load_gather(binc, (r_v,))
            plsc.store_scatter(binc, (r_v,), c_v + 1, mask=lane0)
        return c

    lax.fori_loop(0, nch, pa, 0)

    def pb(r, carry):
        u_v, acc_v = carry
        r_v = splat(r)
        c_v = plsc.load_gather(binc, (r_v,))
        hit = c_v > 0
        plsc.store_scatter(uniq, (u_v, zeros16), r_v + lo,
                           mask=jnp.logical_and(lane0, hit))
        plsc.store_scatter(ust, (u_v,), acc_v,
                           mask=jnp.logical_and(lane0, hit))
        plsc.store_scatter(binc, (r_v,), acc_v, mask=lane0)
        return (u_v + jnp.where(hit, 1, 0).astype(i32), acc_v + c_v)

    u_v, _acc = lax.fori_loop(0, PER_W, pb, (zeros16, zeros16))
    plsc.store_scatter(ust, (u_v,), m_v, mask=lane0)
    plsc.store_scatter(binc, (splat(PER_W),), m_v, mask=lane0)

    def pad(s, c):
        s_v = splat(s)
        cond = jnp.logical_and(lane0, s_v >= u_v)
        plsc.store_scatter(uniq, (s_v, zeros16), splat(lo), mask=cond)
        plsc.store_scatter(ust, (s_v + 1,), m_v, mask=cond)
        return c

    lax.fori_loop(0, NSLOT, pad, 0)

    def pc(g, c):
        x = plsc.load_gather(hits, (g * 16 + lanes,))
        rall = lax.shift_right_logical(x, 13)
        pall = jnp.bitwise_and(x, 0x1FFF)
        for k in range(16):
            r_v = splat(lane(rall, k))
            o_v = plsc.load_gather(binc, (r_v,))
            plsc.store_scatter(opos, (o_v,), splat(lane(pall, k)), mask=lane0)
            plsc.store_scatter(binc, (r_v,), o_v + 1, mask=lane0)
        return c

    lax.fori_loop(0, nch, pc, 0)

    u_s = ext(u_v)
    u_pad = ((u_s + NB - 1) // NB) * NB
    ngr = u_pad // NB

    def g_start(s, b):
        pltpu.async_copy(table_hbm.at[uniq.at[s]], rows.at[b], gsem.at[b])

    def g_wait(s, b):
        pltpu.make_async_copy(
            table_hbm.at[uniq.at[s]], rows.at[b], gsem.at[b]
        ).wait()

    def wr_issue(s, b):
        def inner(h, c):
            pltpu.async_copy(
                rows.at[b], out_hbm.at[pl.ds(rd(opos, h), 1)], osem.at[b]
            )
            return c
        lax.fori_loop(rd(ust, s), rd(ust, s + 1), inner, 0)

    def wr_drain(s, b):
        def inner(h, c):
            pltpu.make_async_copy(
                rows.at[b], out_hbm.at[pl.ds(rd(opos, h), 1)], osem.at[b]
            ).wait()
            return c
        lax.fori_loop(rd(ust, s), rd(ust, s + 1), inner, 0)

    for i in range(AD):
        @pl.when(i < u_pad)
        def _(i=i):
            g_start(i, i)

    def body(g, carry):
        for j in range(NB):
            slot = g * NB + j
            sa = slot + AD
            ba = (j + AD) % NB

            @pl.when(jnp.logical_and(sa < u_pad, sa >= NB))
            def _(sa=sa, ba=ba):
                wr_drain(sa - NB, ba)

            @pl.when(sa < u_pad)
            def _(sa=sa, ba=ba):
                g_start(sa, ba)

            g_wait(slot, j)
            wr_issue(slot, j)
        return carry

    lax.fori_loop(0, ngr, body, 0)

    for j in range(NB):
        @pl.when(u_pad > 0)
        def _(j=j):
            wr_drain(u_pad - NB + j, j)


TC_K = 8


def _tc_gather_body(idx_ref, table_ref, out_ref, sems):
    n = out_ref.shape[0]
    ngr = n // TC_K

    def start(i, j):
        pltpu.make_async_copy(
            table_ref.at[idx_ref[i]], out_ref.at[i], sems.at[j]
        ).start()

    def wait(i, j):
        pltpu.make_async_copy(
            table_ref.at[idx_ref[i]], out_ref.at[i], sems.at[j]
        ).wait()

    for j in range(TC_K):
        start(j, j)

    def body(g, carry):
        for j in range(TC_K):
            i = g * TC_K + j
            wait(i, j)
            start(i + TC_K, j)
        return carry

    lax.fori_loop(0, ngr - 1, body, 0)

    for j in range(TC_K):
        wait((ngr - 1) * TC_K + j, j)


def _tc_gather(idx_flat, table):
    n = idx_flat.shape[0]
    return pl.pallas_call(
        _tc_gather_body,
        in_specs=[
            pl.BlockSpec(memory_space=pltpu.SMEM),
            pl.BlockSpec(memory_space=pltpu.HBM),
        ],
        out_specs=pl.BlockSpec(memory_space=pltpu.HBM),
        out_shape=jax.ShapeDtypeStruct((n, VOCAB), jnp.float32),
        scratch_shapes=[pltpu.SemaphoreType.DMA((TC_K,))],
    )(idx_flat, table)


def _tc_gather_pipelined(idx_flat, table):
    n = idx_flat.shape[0]
    table3 = table.reshape(VOCAB, 1, VOCAB)

    def body(idx_sref, t_ref, o_ref):
        o_ref[...] = t_ref[...]

    out = pl.pallas_call(
        body,
        grid_spec=pltpu.PrefetchScalarGridSpec(
            num_scalar_prefetch=1,
            grid=(n,),
            in_specs=[
                pl.BlockSpec((1, 1, VOCAB), lambda i, idx_ref: (idx_ref[i], 0, 0))
            ],
            out_specs=pl.BlockSpec((1, 1, VOCAB), lambda i, idx_ref: (i, 0, 0)),
        ),
        out_shape=jax.ShapeDtypeStruct((n, 1, VOCAB), jnp.float32),
    )(idx_flat, table3)
    return out.reshape(n, VOCAB)


@functools.partial(
    pl.kernel,
    out_type=jax.ShapeDtypeStruct((N_IDX, VOCAB), jnp.float32),
    mesh=_mesh,
    scratch_types=[
        pltpu.VMEM((NWIN, W), jnp.int32),
        pltpu.VMEM((NBUF, W, VOCAB), jnp.float32),
        pltpu.SemaphoreType.DMA((NBUF,)),
        pltpu.SemaphoreType.DMA((NBUF,)),
    ],
)
def _lookup_readonly(idx_hbm, table_hbm, out_hbm, idx_v, rows_v, gsem, osem):
    wid = lax.axis_index("s") * NC + lax.axis_index("c")
    pltpu.sync_copy(idx_hbm.at[wid], idx_v)

    def g_start(w, b):
        pltpu.async_copy(table_hbm.at[idx_v.at[w]], rows_v.at[b], gsem.at[b])

    def g_wait(w, b):
        pltpu.make_async_copy(
            table_hbm.at[idx_v.at[w]], rows_v.at[b], gsem.at[b]
        ).wait()

    for b in range(NBUF):
        g_start(b, b)

    def body(i, carry):
        for j in range(NBUF):
            w = i * NBUF + j
            g_wait(w, j)
            g_start(w + NBUF, j)
        return carry

    lax.fori_loop(0, NGRP - 1, body, 0)

    for j in range(NBUF):
        g_wait((NGRP - 1) * NBUF + j, j)


@functools.partial(
    pl.kernel,
    out_type=jax.ShapeDtypeStruct((N_IDX, VOCAB), jnp.float32),
    mesh=_mesh,
    scratch_types=[
        pltpu.VMEM((NWIN, W), jnp.int32),
        pltpu.VMEM((NBUF, W, VOCAB), jnp.float32),
        pltpu.SemaphoreType.DMA((NBUF,)),
        pltpu.SemaphoreType.DMA((NBUF,)),
    ],
)
def _lookup_writeonly(idx_hbm, table_hbm, out_hbm, idx_v, rows_v, gsem, osem):
    wid = lax.axis_index("s") * NC + lax.axis_index("c")
    row0 = wid * PER_W
    pltpu.sync_copy(idx_hbm.at[wid], idx_v)

    def o_start(w, b):
        pltpu.async_copy(
            rows_v.at[b], out_hbm.at[pl.ds(row0 + w * W, W)], osem.at[b]
        )

    def o_wait(w, b):
        pltpu.make_async_copy(
            rows_v.at[b], out_hbm.at[pl.ds(row0 + w * W, W)], osem.at[b]
        ).wait()

    for b in range(NBUF):
        o_start(b, b)

    def body(i, carry):
        for j in range(NBUF):
            w = i * NBUF + j
            o_wait(w, j)
            o_start(w + NBUF, j)
        return carry

    lax.fori_loop(0, NGRP - 1, body, 0)

    for j in range(NBUF):
        o_wait((NGRP - 1) * NBUF + j, j)


def kernel(idx, emb_weight):
    out = _lookup_dedup(idx.reshape(N_IDX), emb_weight)
    return out.reshape(idx.shape[0], idx.shape[1], VOCAB)
